# baseline (speedup 1.0000x reference)
import functools

import jax
import jax.numpy as jnp
from jax import lax
from jax.experimental import pallas as pl
from jax.experimental.pallas import tpu as pltpu
from jax.experimental.pallas import tpu_sc as plsc

N = 16384
K = 1024
D = 64
BETA = 0.25
BLK = 2048


def _tc_body(z_ref, e_ref, nearest_ref, zq_ref, loss_ref, et_s, e2_s):
    i = pl.program_id(0)

    @pl.when(i == 0)
    def _():
        et = e_ref[...].T
        et_s[...] = et
        e2_s[...] = jnp.sum(et * et, axis=0, keepdims=True)

    zb = z_ref[...]
    m = jax.lax.dot_general(
        zb, et_s[...], (((1,), (0,)), ((), ())),
        preferred_element_type=jnp.float32)
    s = zb * zb
    a = s[:, 0:8]
    for j in range(1, 8):
        a = a + s[:, 8 * j:8 * j + 8]
    b = a[:, 0:4] + a[:, 4:8]
    c = b[:, 0:2] + b[:, 2:4]
    z2 = c[:, 0:1] + c[:, 1:2]
    t1 = z2 + e2_s[...]
    d2 = t1 - 2.0 * m
    d = jnp.sqrt(jnp.maximum(d2, 0.0))
    dmin_keep = jnp.min(d, axis=1, keepdims=True)
    kiota_f = jax.lax.broadcasted_iota(jnp.int32, (BLK, K), 1).astype(jnp.float32)
    cand = jnp.where(d == dmin_keep, kiota_f, float(K))
    nearest_f = jnp.min(cand, axis=1)
    nearest_ref[0, ...] = nearest_f.astype(jnp.int32).reshape(1, BLK)
    onehot = (kiota_f == nearest_f[:, None]).astype(jnp.float32)
    zq = jax.lax.dot_general(
        onehot, e_ref[...], (((1,), (0,)), ((), ())),
        preferred_element_type=jnp.float32)
    zq_ref[...] = zb + (zq - zb)
    dmin = dmin_keep[:, 0]
    d2min = dmin * dmin

    @pl.when(i == 0)
    def _():
        loss_ref[0, 0] = 0.0

    loss_ref[0, 0] += jnp.sum(d2min)

    @pl.when(i == pl.num_programs(0) - 1)
    def _():
        loss_ref[0, 0] *= (1.0 + BETA) / (N * D)


def _make_sc_gather():
    info = plsc.get_sparse_core_info()
    nc, ns = info.num_cores, info.num_subcores
    nw = nc * ns
    b_per_w = N // nw
    mesh = plsc.VectorSubcoreMesh(core_axis_name="c", subcore_axis_name="s")

    @functools.partial(
        pl.kernel, mesh=mesh,
        out_type=jax.ShapeDtypeStruct((N, 128), jnp.float32),
        scratch_types=[
            pltpu.VMEM((b_per_w,), jnp.int32),
            pltpu.VMEM((b_per_w, 128), jnp.float32),
            pltpu.SemaphoreType.DMA,
        ],
    )
    def sc_gather(idx_hbm, table_hbm, out_hbm, idx_v, rows_v, sem):
        wid = lax.axis_index("s") * nc + lax.axis_index("c")
        base = wid * b_per_w
        pltpu.sync_copy(idx_hbm.at[pl.ds(base, b_per_w)], idx_v)
        pltpu.async_copy(table_hbm.at[idx_v], rows_v, sem).wait()
        pltpu.sync_copy(rows_v, out_hbm.at[pl.ds(base, b_per_w)])

    return sc_gather


_sc_gather = _make_sc_gather()


def kernel(z, embeddings):
    grid = N // BLK
    nearest3, zq, loss_sum = pl.pallas_call(
        _tc_body,
        grid=(grid,),
        in_specs=[
            pl.BlockSpec((BLK, D), lambda i: (i, 0)),
            pl.BlockSpec((K, D), lambda i: (0, 0)),
        ],
        out_specs=[
            pl.BlockSpec((1, 1, BLK), lambda i: (i, 0, 0)),
            pl.BlockSpec((BLK, D), lambda i: (i, 0)),
            pl.BlockSpec(memory_space=pltpu.SMEM),
        ],
        out_shape=[
            jax.ShapeDtypeStruct((grid, 1, BLK), jnp.int32),
            jax.ShapeDtypeStruct((N, D), jnp.float32),
            jax.ShapeDtypeStruct((1, 1), jnp.float32),
        ],
        scratch_shapes=[
            pltpu.VMEM((D, K), jnp.float32),
            pltpu.VMEM((1, K), jnp.float32),
        ],
    )(z, embeddings)
    nearest = nearest3.reshape(N)
    loss = loss_sum[0, 0]
    return (zq, loss, nearest)


# speedup vs baseline: 1.0724x; 1.0724x over previous
import jax
import jax.numpy as jnp
from jax.experimental import pallas as pl
from jax.experimental.pallas import tpu as pltpu

N = 16384
K = 1024
D = 64
BETA = 0.25
BLK = 2048


def _tc_body(z_ref, et_ref, e_ref, z2_ref, e2_ref, nearest_ref, zq_ref, loss_ref):
    i = pl.program_id(0)
    zb = z_ref[...]
    m = jax.lax.dot_general(
        zb, et_ref[...], (((1,), (0,)), ((), ())),
        preferred_element_type=jnp.float32)
    t1 = z2_ref[...] + e2_ref[...]
    d2 = t1 - 2.0 * m
    d2c = jnp.maximum(d2, 0.0)
    m1 = jnp.min(d2c, axis=1, keepdims=True)
    s = jnp.sqrt(m1)
    lo = jax.lax.bitcast_convert_type(m1, jnp.int32)
    hi = lo + 64
    for _ in range(6):
        mid = lo + (hi - lo) // 2
        ok = jnp.sqrt(jax.lax.bitcast_convert_type(mid, jnp.float32)) == s
        lo = jnp.where(ok, mid, lo)
        hi = jnp.where(ok, hi, mid)
    tf = jax.lax.bitcast_convert_type(lo, jnp.float32)
    kiota_f = jax.lax.broadcasted_iota(jnp.int32, (BLK, K), 1).astype(jnp.float32)
    cand = jnp.where(d2c <= tf, kiota_f, float(K))
    nearest_f = jnp.min(cand, axis=1)
    nearest_ref[0, ...] = nearest_f.astype(jnp.int32).reshape(1, BLK)
    onehot = (kiota_f == nearest_f[:, None]).astype(jnp.float32)
    zq = jax.lax.dot_general(
        onehot, e_ref[...], (((1,), (0,)), ((), ())),
        preferred_element_type=jnp.float32)
    zq_ref[...] = zb + (zq - zb)
    dmin = s[:, 0]
    d2min = dmin * dmin

    @pl.when(i == 0)
    def _():
        loss_ref[0, 0] = 0.0

    loss_ref[0, 0] += jnp.sum(d2min)


def kernel(z, embeddings):
    z2 = jnp.sum(z * z, axis=1, keepdims=True)
    e2 = jnp.sum(embeddings * embeddings, axis=1)[None, :]
    et = embeddings.T
    grid = N // BLK
    nearest3, zq, loss_sum = pl.pallas_call(
        _tc_body,
        grid=(grid,),
        in_specs=[
            pl.BlockSpec((BLK, D), lambda i: (i, 0)),
            pl.BlockSpec((D, K), lambda i: (0, 0)),
            pl.BlockSpec((K, D), lambda i: (0, 0)),
            pl.BlockSpec((BLK, 1), lambda i: (i, 0)),
            pl.BlockSpec((1, K), lambda i: (0, 0)),
        ],
        out_specs=[
            pl.BlockSpec((1, 1, BLK), lambda i: (i, 0, 0)),
            pl.BlockSpec((BLK, D), lambda i: (i, 0)),
            pl.BlockSpec(memory_space=pltpu.SMEM),
        ],
        out_shape=[
            jax.ShapeDtypeStruct((grid, 1, BLK), jnp.int32),
            jax.ShapeDtypeStruct((N, D), jnp.float32),
            jax.ShapeDtypeStruct((1, 1), jnp.float32),
        ],
    )(z, et, embeddings, z2, e2)
    nearest = nearest3.reshape(N)
    loss = loss_sum[0, 0] * ((1.0 + BETA) / (N * D))
    return (zq, loss, nearest)


# speedup vs baseline: 1.1372x; 1.0604x over previous
import functools

import jax
import jax.numpy as jnp
from jax import lax
from jax.experimental import pallas as pl
from jax.experimental.pallas import tpu as pltpu
from jax.experimental.pallas import tpu_sc as plsc

N = 16384
K = 1024
D = 64
BETA = 0.25
BLK = 2048
NCHUNK = 2
CH = N // NCHUNK


def _tc_body(z_ref, et_ref, z2_ref, e2_ref, nearest_ref, loss_ref):
    i = pl.program_id(0)
    zb = z_ref[...]
    m = jax.lax.dot_general(
        zb, et_ref[...], (((1,), (0,)), ((), ())),
        preferred_element_type=jnp.float32)
    t1 = z2_ref[...] + e2_ref[...]
    d2 = t1 - 2.0 * m
    d = jnp.sqrt(jnp.maximum(d2, 0.0))
    dmin_keep = jnp.min(d, axis=1, keepdims=True)
    kiota_f = jax.lax.broadcasted_iota(jnp.int32, (BLK, K), 1).astype(jnp.float32)
    cand = jnp.where(d == dmin_keep, kiota_f, float(K))
    nearest_f = jnp.min(cand, axis=1)
    nearest_ref[0, ...] = nearest_f.astype(jnp.int32).reshape(1, BLK)
    dmin = dmin_keep[:, 0]
    d2min = dmin * dmin

    @pl.when(i == 0)
    def _():
        loss_ref[0, 0] = 0.0

    loss_ref[0, 0] += jnp.sum(d2min)


def _tc_chunk(zc, et, z2c, e2):
    grid = CH // BLK
    return pl.pallas_call(
        _tc_body,
        grid=(grid,),
        in_specs=[
            pl.BlockSpec((BLK, D), lambda i: (i, 0)),
            pl.BlockSpec((D, K), lambda i: (0, 0)),
            pl.BlockSpec((BLK, 1), lambda i: (i, 0)),
            pl.BlockSpec((1, K), lambda i: (0, 0)),
        ],
        out_specs=[
            pl.BlockSpec((1, 1, BLK), lambda i: (i, 0, 0)),
            pl.BlockSpec(memory_space=pltpu.SMEM),
        ],
        out_shape=[
            jax.ShapeDtypeStruct((grid, 1, BLK), jnp.int32),
            jax.ShapeDtypeStruct((1, 1), jnp.float32),
        ],
    )(zc, et, z2c, e2)


def _make_sc_gather(nrows):
    info = plsc.get_sparse_core_info()
    nc, ns = info.num_cores, info.num_subcores
    nw = nc * ns
    b_per_w = nrows // nw
    mesh = plsc.VectorSubcoreMesh(core_axis_name="c", subcore_axis_name="s")

    @functools.partial(
        pl.kernel, mesh=mesh,
        out_type=jax.ShapeDtypeStruct((nrows, 128), jnp.float32),
        scratch_types=[
            pltpu.VMEM((b_per_w,), jnp.int32),
            pltpu.VMEM((b_per_w, 128), jnp.float32),
            pltpu.SemaphoreType.DMA,
        ],
    )
    def sc_gather(idx_hbm, table_hbm, out_hbm, idx_v, rows_v, sem):
        wid = lax.axis_index("s") * nc + lax.axis_index("c")
        base = wid * b_per_w
        pltpu.sync_copy(idx_hbm.at[pl.ds(base, b_per_w)], idx_v)
        pltpu.async_copy(table_hbm.at[idx_v], rows_v, sem).wait()
        pltpu.sync_copy(rows_v, out_hbm.at[pl.ds(base, b_per_w)])

    return sc_gather


_sc_gather = _make_sc_gather(CH)


def kernel(z, embeddings):
    z2 = jnp.sum(z * z, axis=1, keepdims=True)
    e2 = jnp.sum(embeddings * embeddings, axis=1)[None, :]
    et = embeddings.T
    table128 = jnp.concatenate(
        [embeddings, jnp.zeros((K, 128 - D), jnp.float32)], axis=1)
    nearests, zqs, loss = [], [], jnp.zeros((), jnp.float32)
    for c in range(NCHUNK):
        r0 = c * CH
        nearest3, loss_sum = _tc_chunk(
            jax.lax.slice(z, (r0, 0), (r0 + CH, D)), et,
            jax.lax.slice(z2, (r0, 0), (r0 + CH, 1)), e2)
        nearest = nearest3.reshape(CH)
        zqs.append(_sc_gather(nearest, table128))
        nearests.append(nearest)
        loss = loss + loss_sum[0, 0]
    nearest = jnp.concatenate(nearests)
    zq = jnp.concatenate(zqs)[:, :D]
    loss = loss * ((1.0 + BETA) / (N * D))
    return (zq, loss, nearest)


# speedup vs baseline: 1.2425x; 1.0926x over previous
import functools

import jax
import jax.numpy as jnp
from jax import lax
from jax.experimental import pallas as pl
from jax.experimental.pallas import tpu as pltpu
from jax.experimental.pallas import tpu_sc as plsc

N = 16384
K = 1024
D = 64
BETA = 0.25
BLK = 2048


def _tc_body(z_ref, et_ref, z2_ref, e2_ref, nearest_ref, loss_ref):
    i = pl.program_id(0)
    zb = z_ref[...]
    m = jax.lax.dot_general(
        zb, et_ref[...], (((1,), (0,)), ((), ())),
        preferred_element_type=jnp.float32)
    t1 = z2_ref[...] + e2_ref[...]
    d2 = t1 - 2.0 * m
    d = jnp.sqrt(jnp.maximum(d2, 0.0))
    dmin_keep = jnp.min(d, axis=1, keepdims=True)
    kiota_f = jax.lax.broadcasted_iota(jnp.int32, (BLK, K), 1).astype(jnp.float32)
    cand = jnp.where(d == dmin_keep, kiota_f, float(K))
    nearest_f = jnp.min(cand, axis=1)
    nearest_ref[0, ...] = nearest_f.astype(jnp.int32).reshape(1, BLK)
    dmin = dmin_keep[:, 0]
    d2min = dmin * dmin

    @pl.when(i == 0)
    def _():
        loss_ref[0, 0] = 0.0

    loss_ref[0, 0] += jnp.sum(d2min)


def _make_sc_gather():
    info = plsc.get_sparse_core_info()
    nc, ns = info.num_cores, info.num_subcores
    nw = nc * ns
    b_per_w = N // nw
    mesh = plsc.VectorSubcoreMesh(core_axis_name="c", subcore_axis_name="s")

    @functools.partial(
        pl.kernel, mesh=mesh,
        out_type=jax.ShapeDtypeStruct((N, 128), jnp.float32),
        scratch_types=[
            pltpu.VMEM((b_per_w,), jnp.int32),
            pltpu.VMEM((b_per_w, 128), jnp.float32),
            pltpu.SemaphoreType.DMA,
        ],
    )
    def sc_gather(idx_hbm, table_hbm, out_hbm, idx_v, rows_v, sem):
        wid = lax.axis_index("s") * nc + lax.axis_index("c")
        base = wid * b_per_w
        pltpu.sync_copy(idx_hbm.at[pl.ds(base, b_per_w)], idx_v)
        pltpu.async_copy(table_hbm.at[idx_v], rows_v, sem).wait()
        pltpu.sync_copy(rows_v, out_hbm.at[pl.ds(base, b_per_w)])

    return sc_gather


_sc_gather = _make_sc_gather()


def kernel(z, embeddings):
    z2 = jnp.sum(z * z, axis=1, keepdims=True)
    e2 = jnp.sum(embeddings * embeddings, axis=1)[None, :]
    et = embeddings.T
    grid = N // BLK
    nearest3, loss_sum = pl.pallas_call(
        _tc_body,
        grid=(grid,),
        in_specs=[
            pl.BlockSpec((BLK, D), lambda i: (i, 0)),
            pl.BlockSpec((D, K), lambda i: (0, 0)),
            pl.BlockSpec((BLK, 1), lambda i: (i, 0)),
            pl.BlockSpec((1, K), lambda i: (0, 0)),
        ],
        out_specs=[
            pl.BlockSpec((1, 1, BLK), lambda i: (i, 0, 0)),
            pl.BlockSpec(memory_space=pltpu.SMEM),
        ],
        out_shape=[
            jax.ShapeDtypeStruct((grid, 1, BLK), jnp.int32),
            jax.ShapeDtypeStruct((1, 1), jnp.float32),
        ],
    )(z, et, z2, e2)
    nearest = nearest3.reshape(N)
    table128 = jnp.concatenate(
        [embeddings, jnp.zeros((K, 128 - D), jnp.float32)], axis=1)
    zq = _sc_gather(nearest, table128)[:, :D]
    loss = loss_sum[0, 0] * ((1.0 + BETA) / (N * D))
    return (zq, loss, nearest)
